# Initial kernel scaffold; baseline (speedup 1.0000x reference)
import functools

import jax
import jax.numpy as jnp
from jax import lax
from jax.experimental import pallas as pl
from jax.experimental.pallas import tpu as pltpu
from jax.experimental.pallas import tpu_sc as plsc

B = 16384
L = 50
D = 32
NC, NS = 2, 16
NW = NC * NS
BAGS_W = B // NW
CB = 16
STEPS = BAGS_W // CB
IPC = CB * L
GN = 100
NG = IPC // GN
HALF = 16


def _bag_kernel(x_hbm, w_hbm, out_hbm, idx_v, rows_v, out_v, sem):
    cid = lax.axis_index("c")
    sid = lax.axis_index("s")
    wid = sid * NC + cid
    base_bag = wid * BAGS_W

    def step(s, carry):
        bag0 = base_bag + s * CB
        row0 = bag0 * L // GN
        pltpu.sync_copy(x_hbm.at[pl.ds(row0, NG)], idx_v)
        copies = [
            pltpu.async_copy(
                w_hbm.at[idx_v.at[j]],
                rows_v.at[pl.ds(j * GN, GN)],
                sem,
            )
            for j in range(NG)
        ]
        for c in copies:
            c.wait()

        def bag_body(b, carry2):
            def row_body(r, acc):
                a0, a1 = acc
                i = b * L + r
                v0 = rows_v[i, pl.ds(0, HALF)]
                v1 = rows_v[i, pl.ds(HALF, HALF)]
                return (a0 + jnp.sign(v0), a1 + jnp.sign(v1))

            zero = jnp.zeros((HALF,), jnp.float32)
            a0, a1 = lax.fori_loop(0, L, row_body, (zero, zero))
            out_v[b, pl.ds(0, HALF)] = jnp.sign(a0)
            out_v[b, pl.ds(HALF, HALF)] = jnp.sign(a1)
            return carry2

        lax.fori_loop(0, CB, bag_body, 0)
        pltpu.sync_copy(out_v, out_hbm.at[pl.ds(bag0, CB)])
        return carry

    lax.fori_loop(0, STEPS, step, 0)


@jax.jit
def kernel(x, weight):
    x2d = x.astype(jnp.int32).reshape(B * L // GN, GN)
    mesh = plsc.VectorSubcoreMesh(core_axis_name="c", subcore_axis_name="s")
    f = pl.kernel(
        _bag_kernel,
        out_type=jax.ShapeDtypeStruct((B, D), jnp.float32),
        mesh=mesh,
        scratch_types=[
            pltpu.VMEM((NG, GN), jnp.int32),
            pltpu.VMEM((IPC, D), jnp.float32),
            pltpu.VMEM((CB, D), jnp.float32),
            pltpu.SemaphoreType.DMA,
        ],
    )
    return f(x2d, weight)


# speedup vs baseline: 2.5601x; 2.5601x over previous
import functools

import jax
import jax.numpy as jnp
from jax import lax
from jax.experimental import pallas as pl
from jax.experimental.pallas import tpu as pltpu
from jax.experimental.pallas import tpu_sc as plsc

B = 16384
L = 50
D = 32
NC, NS = 2, 16
NW = NC * NS
BAGS_W = B // NW
CB = 16
STEPS = BAGS_W // CB
IPC = CB * L
GN = 100
NG = IPC // GN
HALF = 16


def _bag_kernel(x_hbm, w_hbm, out_hbm, idx_v, rows_v, out_v, sem):
    cid = lax.axis_index("c")
    sid = lax.axis_index("s")
    wid = sid * NC + cid
    base_bag = wid * BAGS_W

    def step(s, carry):
        bag0 = base_bag + s * CB
        row0 = pl.multiple_of(bag0 * L // GN, 8)
        pltpu.sync_copy(x_hbm.at[pl.ds(row0, NG)], idx_v)
        copies = [
            pltpu.async_copy(
                w_hbm.at[idx_v.at[j]],
                rows_v.at[pl.ds(j * GN, GN)],
                sem,
            )
            for j in range(NG)
        ]
        for c in copies:
            c.wait()

        def bag_body(b, carry2):
            def row_body(r, acc):
                a0, a1 = acc
                i = b * L + r
                v0 = rows_v[i, pl.ds(0, HALF)]
                v1 = rows_v[i, pl.ds(HALF, HALF)]
                return (a0 + jnp.sign(v0), a1 + jnp.sign(v1))

            zero = jnp.zeros((HALF,), jnp.float32)
            a0, a1 = lax.fori_loop(0, L, row_body, (zero, zero))
            out_v[b, pl.ds(0, HALF)] = jnp.sign(a0)
            out_v[b, pl.ds(HALF, HALF)] = jnp.sign(a1)
            return carry2

        lax.fori_loop(0, CB, bag_body, 0)
        pltpu.sync_copy(out_v, out_hbm.at[pl.ds(pl.multiple_of(bag0, 8), CB)])
        return carry

    lax.fori_loop(0, STEPS, step, 0)


@jax.jit
def kernel(x, weight):
    x2d = x.astype(jnp.int32).reshape(B * L // GN, GN)
    mesh = plsc.VectorSubcoreMesh(core_axis_name="c", subcore_axis_name="s")
    f = pl.kernel(
        _bag_kernel,
        out_type=jax.ShapeDtypeStruct((B, D), jnp.float32),
        mesh=mesh,
        scratch_types=[
            pltpu.VMEM((NG, GN), jnp.int32),
            pltpu.VMEM((IPC, D), jnp.float32),
            pltpu.VMEM((CB, D), jnp.float32),
            pltpu.SemaphoreType.DMA,
        ],
        compiler_params=pltpu.CompilerParams(use_tc_tiling_on_sc=False),
    )
    return f(x2d, weight)


# speedup vs baseline: 2.8365x; 1.1079x over previous
import functools

import jax
import jax.numpy as jnp
from jax import lax
from jax.experimental import pallas as pl
from jax.experimental.pallas import tpu as pltpu
from jax.experimental.pallas import tpu_sc as plsc

B = 16384
L = 50
D = 32
NC, NS = 2, 16
NW = NC * NS
BAGS_W = B // NW
CB = 16
STEPS = BAGS_W // CB
IPC = CB * L
GN = 100
NG = IPC // GN
HALF = 16


def _bag_kernel(x_hbm, w_hbm, out_hbm, idx_v, rows_v, out_v, sem0, sem1):
    cid = lax.axis_index("c")
    sid = lax.axis_index("s")
    wid = sid * NC + cid
    base_bag = wid * BAGS_W
    sems = (sem0, sem1)

    def fire(s, buf):
        bag0 = base_bag + s * CB
        row0 = pl.multiple_of(bag0 * L // GN, 8)
        pltpu.sync_copy(x_hbm.at[pl.ds(row0, NG)], idx_v.at[buf])
        for j in range(NG):
            pltpu.async_copy(
                w_hbm.at[idx_v.at[buf, j]],
                rows_v.at[buf, pl.ds(j * GN, GN)],
                sems[buf],
            )

    def drain(buf):
        pltpu.make_async_copy(
            w_hbm.at[pl.ds(0, IPC)], rows_v.at[buf], sems[buf]
        ).wait()

    def compute(s, buf):
        bag0 = base_bag + s * CB

        def bag_body(b, carry):
            base = b * L
            zero = jnp.zeros((HALF,), jnp.float32)
            acc = [zero, zero, zero, zero]
            for r in range(L):
                v0 = rows_v[buf, base + r, pl.ds(0, HALF)]
                v1 = rows_v[buf, base + r, pl.ds(HALF, HALF)]
                acc[r % 2] = acc[r % 2] + jnp.sign(v0)
                acc[2 + r % 2] = acc[2 + r % 2] + jnp.sign(v1)
            out_v[b, pl.ds(0, HALF)] = jnp.sign(acc[0] + acc[1])
            out_v[b, pl.ds(HALF, HALF)] = jnp.sign(acc[2] + acc[3])
            return carry

        lax.fori_loop(0, CB, bag_body, 0)
        pltpu.sync_copy(out_v, out_hbm.at[pl.ds(pl.multiple_of(bag0, 8), CB)])

    fire(0, 0)

    def body(h, carry):
        s0 = 2 * h
        fire(s0 + 1, 1)
        drain(0)
        compute(s0, 0)

        @pl.when(s0 + 2 < STEPS)
        def _():
            fire(s0 + 2, 0)

        drain(1)
        compute(s0 + 1, 1)
        return carry

    lax.fori_loop(0, STEPS // 2, body, 0)


@jax.jit
def kernel(x, weight):
    x2d = x.astype(jnp.int32).reshape(B * L // GN, GN)
    mesh = plsc.VectorSubcoreMesh(core_axis_name="c", subcore_axis_name="s")
    f = pl.kernel(
        _bag_kernel,
        out_type=jax.ShapeDtypeStruct((B, D), jnp.float32),
        mesh=mesh,
        scratch_types=[
            pltpu.VMEM((2, NG, GN), jnp.int32),
            pltpu.VMEM((2, IPC, D), jnp.float32),
            pltpu.VMEM((CB, D), jnp.float32),
            pltpu.SemaphoreType.DMA,
            pltpu.SemaphoreType.DMA,
        ],
        compiler_params=pltpu.CompilerParams(use_tc_tiling_on_sc=False),
    )
    return f(x2d, weight)
